# Initial kernel scaffold; baseline (speedup 1.0000x reference)
import jax
import jax.numpy as jnp
from jax.experimental import pallas as pl


def kernel(user_idx, item_idx, embedding, alpha_head2tail, alpha_tail2head, edge_index, edge_type, edge_norm):
    raise NotImplementedError("write your pallas kernel here")


# speedup vs baseline: 1.2565x; 1.2565x over previous
import jax
import jax.numpy as jnp
from jax import lax
from jax.experimental import pallas as pl
from jax.experimental.pallas import tpu as pltpu
from jax.experimental.pallas import tpu_sc as plsc

N_NODE = 10000
N_USER = 6000
D = 128
NE = 320000
NTILE = 16
TPD = 8
EPT = NE // TPD
CK = 80
NCHUNK = EPT // CK
ROWS_PT = N_NODE // NTILE
ZR = 25
BATCH = 4096
OPT = BATCH // TPD
OCK = 64


def _body(user_idx, item_idx, emb, h_arr, t_arr, etype, enorm, acat,
          out_u, out_i, e1, e2, e3,
          acc, src_v, dst_v, nrm_v, typ_v, cof_v, rows_v, zer_v,
          acat_v, oidx_v, orow_v, oacc_v, sem):
    tid = lax.axis_index("s")
    dir_a = tid < TPD
    lane = jnp.where(dir_a, tid, tid - TPD)
    ebase = lane * EPT

    pltpu.sync_copy(acat, acat_v)
    aoff = jnp.where(dir_a, 0, 16).astype(jnp.int32)

    for r in range(ZR):
        for u in range(8):
            zer_v[r, pl.ds(u * 16, 16)] = jnp.zeros((16,), jnp.float32)

    def scatter_dir(src_hbm, dst_hbm, esrc):
        def cbody(i, _):
            base = ebase + i * CK
            pltpu.sync_copy(src_hbm.at[pl.ds(base, CK)], src_v)
            pltpu.sync_copy(dst_hbm.at[pl.ds(base, CK)], dst_v)
            pltpu.sync_copy(enorm.at[pl.ds(base, CK)], nrm_v)
            pltpu.sync_copy(etype.at[pl.ds(base, CK)], typ_v)
            pltpu.async_copy(esrc.at[src_v], rows_v, sem).wait()

            def kbody(j, _):
                vt = typ_v[pl.ds(j * 16, 16)]
                vn = nrm_v[pl.ds(j * 16, 16)]
                va = plsc.load_gather(acat_v, [vt + aoff])
                cof_v[pl.ds(j * 16, 16)] = vn * va
                return 0

            lax.fori_loop(0, CK // 16, kbody, 0)

            def rbody(r, _):
                cs = plsc.load_gather(
                    cof_v, [jnp.full((16,), r, dtype=jnp.int32)])
                for u in range(8):
                    rows_v[r, pl.ds(u * 16, 16)] = (
                        rows_v[r, pl.ds(u * 16, 16)] * cs)
                return 0

            lax.fori_loop(0, CK, rbody, 0)
            pltpu.sync_copy(rows_v, acc.at[dst_v], add=True)
            return 0

        lax.fori_loop(0, NCHUNK, cbody, 0)

    for l, (esrc, edst) in enumerate(((emb, e1), (e1, e2), (e2, e3))):
        for z in range(ROWS_PT // ZR):
            pltpu.sync_copy(zer_v, acc.at[pl.ds(tid * ROWS_PT + z * ZR, ZR)])
        plsc.subcore_barrier()
        pl.when(dir_a)(lambda: scatter_dir(t_arr, h_arr, esrc))
        pl.when(jnp.logical_not(dir_a))(lambda: scatter_dir(h_arr, t_arr, esrc))
        plsc.subcore_barrier()
        pltpu.sync_copy(acc.at[pl.ds(tid * ROWS_PT, ROWS_PT)],
                        edst.at[pl.ds(tid * ROWS_PT, ROWS_PT)])
        plsc.subcore_barrier()

    def out_dir(idx_hbm, out_hbm, row_off):
        def obody(q, _):
            ob = lane * OPT + q * OCK
            pltpu.sync_copy(idx_hbm.at[pl.ds(ob, OCK)], oidx_v)
            if row_off:
                for m in range(OCK // 16):
                    oidx_v[pl.ds(m * 16, 16)] = (
                        oidx_v[pl.ds(m * 16, 16)] + jnp.int32(row_off))
            pltpu.async_copy(emb.at[oidx_v], oacc_v, sem).wait()
            for tbl in (e1, e2, e3):
                pltpu.async_copy(tbl.at[oidx_v], orow_v, sem).wait()

                def abody(j, _):
                    for u in range(8):
                        oacc_v[j, pl.ds(u * 16, 16)] = (
                            oacc_v[j, pl.ds(u * 16, 16)]
                            + orow_v[j, pl.ds(u * 16, 16)])
                    return 0

                lax.fori_loop(0, OCK, abody, 0)

            def sbody(j, _):
                for u in range(8):
                    oacc_v[j, pl.ds(u * 16, 16)] = (
                        oacc_v[j, pl.ds(u * 16, 16)] * 0.25)
                return 0

            lax.fori_loop(0, OCK, sbody, 0)
            pltpu.sync_copy(oacc_v, out_hbm.at[pl.ds(ob, OCK)])
            return 0

        lax.fori_loop(0, OPT // OCK, obody, 0)

    pl.when(dir_a)(lambda: out_dir(user_idx, out_u, 0))
    pl.when(jnp.logical_not(dir_a))(lambda: out_dir(item_idx, out_i, N_USER))


@jax.jit
def kernel(user_idx, item_idx, embedding, alpha_head2tail, alpha_tail2head,
           edge_index, edge_type, edge_norm):
    h_arr = edge_index[0]
    t_arr = edge_index[1]
    acat = jnp.concatenate([alpha_tail2head, alpha_head2tail])

    f32 = jnp.float32
    call = pl.kernel(
        _body,
        out_type=(
            jax.ShapeDtypeStruct((BATCH, D), f32),
            jax.ShapeDtypeStruct((BATCH, D), f32),
            jax.ShapeDtypeStruct((N_NODE, D), f32),
            jax.ShapeDtypeStruct((N_NODE, D), f32),
            jax.ShapeDtypeStruct((N_NODE, D), f32),
        ),
        mesh=plsc.VectorSubcoreMesh(
            core_axis_name="c", subcore_axis_name="s", num_cores=1),
        scratch_types=(
            pltpu.VMEM_SHARED((N_NODE, D), f32),
            pltpu.VMEM((CK,), jnp.int32),
            pltpu.VMEM((CK,), jnp.int32),
            pltpu.VMEM((CK,), f32),
            pltpu.VMEM((CK,), jnp.int32),
            pltpu.VMEM((CK,), f32),
            pltpu.VMEM((CK, D), f32),
            pltpu.VMEM((ZR, D), f32),
            pltpu.VMEM((32,), f32),
            pltpu.VMEM((OCK,), jnp.int32),
            pltpu.VMEM((OCK, D), f32),
            pltpu.VMEM((OCK, D), f32),
            pltpu.SemaphoreType.DMA,
        ),
        compiler_params=pltpu.CompilerParams(use_tc_tiling_on_sc=False,
                                             needs_layout_passes=False),
    )
    out_u, out_i, _, _, _ = call(user_idx, item_idx, embedding, h_arr, t_arr,
                                 edge_type, edge_norm, acat)
    return out_u, out_i


# speedup vs baseline: 3.9452x; 3.1399x over previous
import jax
import jax.numpy as jnp
from jax import lax
from jax.experimental import pallas as pl
from jax.experimental.pallas import tpu as pltpu
from jax.experimental.pallas import tpu_sc as plsc

N_NODE = 10000
N_USER = 6000
D = 128
NE = 320000
NM = 2 * NE
NTILE = 16
MPT = NM // NTILE
CK = 64
NSUB = 25
SCM = CK * NSUB
NSUP = MPT // SCM
NB = 3
ROWS_PT = N_NODE // NTILE
ZR = 25
NOUT = 8192
OPT = NOUT // NTILE
OCK = 32


def _body(oidx_all, emb, src2d, dst2d, typall, nrmall, acat,
          out_all, e1, e2, e3,
          acc, src2_v, dst2_v, nrm2_v, typ2_v, cof2_v, rows_v, zer_v,
          acat_v, oidx_v, orow_v, oacc_v,
          sem_m, sem_o, sems_g, sems_s):
    tid = lax.axis_index("s")
    ebase = tid * MPT
    cbase = tid * (MPT // CK)

    pltpu.sync_copy(acat, acat_v)
    aoff = jnp.where(tid < (NTILE // 2), 0, 16).astype(jnp.int32)

    for r in range(ZR):
        for u in range(8):
            zer_v[r, pl.ds(u * 16, 16)] = jnp.zeros((16,), jnp.float32)

    def scatter_layer(esrc):
        def sbody(s, _):
            mb = ebase + s * SCM
            cb = cbase + s * NSUB
            d1 = pltpu.async_copy(src2d.at[pl.ds(cb, NSUB)], src2_v, sem_m)
            d2 = pltpu.async_copy(dst2d.at[pl.ds(cb, NSUB)], dst2_v, sem_m)
            d3 = pltpu.async_copy(nrmall.at[pl.ds(mb, SCM)], nrm2_v, sem_m)
            d4 = pltpu.async_copy(typall.at[pl.ds(mb, SCM)], typ2_v, sem_m)
            d1.wait(); d2.wait(); d3.wait(); d4.wait()

            def kbody(j, _):
                vt = typ2_v[pl.ds(j * 16, 16)]
                vn = nrm2_v[pl.ds(j * 16, 16)]
                va = plsc.load_gather(acat_v, [vt + aoff])
                cof2_v[pl.ds(j * 16, 16)] = vn * va
                return 0

            lax.fori_loop(0, SCM // 16, kbody, 0)

            def scale(j):
                b = j % NB

                def rbody(r, _):
                    cs = plsc.load_gather(
                        cof2_v, [jnp.full((16,), j * CK + r, dtype=jnp.int32)])
                    for u in range(8):
                        rows_v[b][r, pl.ds(u * 16, 16)] = (
                            rows_v[b][r, pl.ds(u * 16, 16)] * cs)
                    return 0

                lax.fori_loop(0, CK, rbody, 0)

            descs_g = [None] * NB
            descs_s = [None] * NB
            for j in range(NSUB):
                b = j % NB
                if descs_s[b] is not None:
                    descs_s[b].wait()
                descs_g[b] = pltpu.async_copy(
                    esrc.at[src2_v.at[j]], rows_v[b], sems_g[b])
                if j >= 1:
                    pb = (j - 1) % NB
                    descs_g[pb].wait()
                    scale(j - 1)
                    descs_s[pb] = pltpu.async_copy(
                        rows_v[pb], acc.at[dst2_v.at[j - 1]], sems_s[pb],
                        add=True)
            lb = (NSUB - 1) % NB
            descs_g[lb].wait()
            scale(NSUB - 1)
            descs_s[lb] = pltpu.async_copy(
                rows_v[lb], acc.at[dst2_v.at[NSUB - 1]], sems_s[lb], add=True)
            for b in range(NB):
                if descs_s[b] is not None:
                    descs_s[b].wait()
            return 0

        lax.fori_loop(0, NSUP, sbody, 0)

    for esrc, edst in ((emb, e1), (e1, e2), (e2, e3)):
        def zbody(z, _):
            pltpu.sync_copy(zer_v, acc.at[pl.ds(tid * ROWS_PT + z * ZR, ZR)])
            return 0

        lax.fori_loop(0, ROWS_PT // ZR, zbody, 0)
        plsc.subcore_barrier()
        scatter_layer(esrc)
        plsc.subcore_barrier()
        pltpu.sync_copy(acc.at[pl.ds(tid * ROWS_PT, ROWS_PT)],
                        edst.at[pl.ds(tid * ROWS_PT, ROWS_PT)])
        plsc.subcore_barrier()

    def obody(q, _):
        ob = tid * OPT + q * OCK
        pltpu.sync_copy(oidx_all.at[pl.ds(ob, OCK)], oidx_v)
        pltpu.async_copy(emb.at[oidx_v], oacc_v, sem_o).wait()
        for tbl in (e1, e2, e3):
            pltpu.async_copy(tbl.at[oidx_v], orow_v, sem_o).wait()

            def abody(j, _):
                for u in range(8):
                    oacc_v[j, pl.ds(u * 16, 16)] = (
                        oacc_v[j, pl.ds(u * 16, 16)]
                        + orow_v[j, pl.ds(u * 16, 16)])
                return 0

            lax.fori_loop(0, OCK, abody, 0)

        def sbody(j, _):
            for u in range(8):
                oacc_v[j, pl.ds(u * 16, 16)] = (
                    oacc_v[j, pl.ds(u * 16, 16)] * 0.25)
            return 0

        lax.fori_loop(0, OCK, sbody, 0)
        pltpu.sync_copy(oacc_v, out_all.at[pl.ds(ob, OCK)])
        return 0

    lax.fori_loop(0, OPT // OCK, obody, 0)


@jax.jit
def kernel(user_idx, item_idx, embedding, alpha_head2tail, alpha_tail2head,
           edge_index, edge_type, edge_norm):
    h_arr = edge_index[0]
    t_arr = edge_index[1]
    src2d = jnp.concatenate([t_arr, h_arr]).reshape(NM // CK, CK)
    dst2d = jnp.concatenate([h_arr, t_arr]).reshape(NM // CK, CK)
    typall = jnp.concatenate([edge_type, edge_type])
    nrmall = jnp.concatenate([edge_norm, edge_norm])
    acat = jnp.concatenate([alpha_tail2head, alpha_head2tail])
    oidx_all = jnp.concatenate([user_idx, item_idx + N_USER])

    f32 = jnp.float32
    call = pl.kernel(
        _body,
        out_type=(
            jax.ShapeDtypeStruct((NOUT, D), f32),
            jax.ShapeDtypeStruct((N_NODE, D), f32),
            jax.ShapeDtypeStruct((N_NODE, D), f32),
            jax.ShapeDtypeStruct((N_NODE, D), f32),
        ),
        mesh=plsc.VectorSubcoreMesh(
            core_axis_name="c", subcore_axis_name="s", num_cores=1),
        scratch_types=(
            pltpu.VMEM_SHARED((N_NODE, D), f32),
            pltpu.VMEM((NSUB, CK), jnp.int32),
            pltpu.VMEM((NSUB, CK), jnp.int32),
            pltpu.VMEM((SCM,), f32),
            pltpu.VMEM((SCM,), jnp.int32),
            pltpu.VMEM((SCM,), f32),
            tuple(pltpu.VMEM((CK, D), f32) for _ in range(NB)),
            pltpu.VMEM((ZR, D), f32),
            pltpu.VMEM((32,), f32),
            pltpu.VMEM((OCK,), jnp.int32),
            pltpu.VMEM((OCK, D), f32),
            pltpu.VMEM((OCK, D), f32),
            pltpu.SemaphoreType.DMA,
            pltpu.SemaphoreType.DMA,
            tuple(pltpu.SemaphoreType.DMA for _ in range(NB)),
            tuple(pltpu.SemaphoreType.DMA for _ in range(NB)),
        ),
        compiler_params=pltpu.CompilerParams(use_tc_tiling_on_sc=False,
                                             needs_layout_passes=False),
    )
    out_all, _, _, _ = call(oidx_all, embedding, src2d, dst2d, typall,
                            nrmall, acat)
    return out_all[:4096], out_all[4096:]


# speedup vs baseline: 7.5044x; 1.9022x over previous
import jax
import jax.numpy as jnp
from jax import lax
from jax.experimental import pallas as pl
from jax.experimental.pallas import tpu as pltpu
from jax.experimental.pallas import tpu_sc as plsc

N_NODE = 10000
N_USER = 6000
D = 128
NE = 320000
NM = 2 * NE
NTILE = 16
NC = 2
MPT = NE // NTILE
CK = 80
NSUB = 25
SCM = CK * NSUB
NSUP = MPT // SCM
NB = 3
ROWS_PT = N_NODE // NTILE
ZR = 25
NOUT = 8192
OPT = NOUT // (NC * NTILE)
OCK = 32
CW_HI = 313
CW_LO = 312

_params = pltpu.CompilerParams(use_tc_tiling_on_sc=False,
                               needs_layout_passes=False)
_mesh = plsc.VectorSubcoreMesh(core_axis_name="c", subcore_axis_name="s",
                               num_cores=NC)
f32 = jnp.float32
i32 = jnp.int32


def _scatter_body(esrc, src2d, dst2d, typall, nrmall, acat, p_out,
                  acc, src2_v, dst2_v, nrm2_v, typ2_v, cof2_v, rows_v,
                  zer_v, acat_v, sem_m, sems_g, sems_s):
    cid = lax.axis_index("c")
    sid = lax.axis_index("s")
    ebase = cid * NE + sid * MPT
    cbase = cid * (NE // CK) + sid * (MPT // CK)

    pltpu.sync_copy(acat, acat_v)
    aoff = jnp.where(cid == 0, 0, 16).astype(i32)

    for r in range(ZR):
        for u in range(8):
            zer_v[r, pl.ds(u * 16, 16)] = jnp.zeros((16,), f32)

    def zbody(z, _):
        pltpu.sync_copy(zer_v, acc.at[pl.ds(sid * ROWS_PT + z * ZR, ZR)])
        return 0

    lax.fori_loop(0, ROWS_PT // ZR, zbody, 0)
    plsc.subcore_barrier()

    def sbody(s, _):
        mb = ebase + s * SCM
        cb = cbase + s * NSUB
        d1 = pltpu.async_copy(src2d.at[pl.ds(cb, NSUB)], src2_v, sem_m)
        d2 = pltpu.async_copy(dst2d.at[pl.ds(cb, NSUB)], dst2_v, sem_m)
        d3 = pltpu.async_copy(nrmall.at[pl.ds(mb, SCM)], nrm2_v, sem_m)
        d4 = pltpu.async_copy(typall.at[pl.ds(mb, SCM)], typ2_v, sem_m)
        d1.wait(); d2.wait(); d3.wait(); d4.wait()

        def kbody(j, _):
            vt = typ2_v[pl.ds(j * 16, 16)]
            vn = nrm2_v[pl.ds(j * 16, 16)]
            va = plsc.load_gather(acat_v, [vt + aoff])
            cof2_v[pl.ds(j * 16, 16)] = vn * va
            return 0

        lax.fori_loop(0, SCM // 16, kbody, 0)

        def scale(j):
            b = j % NB

            def rbody(r, _):
                cs = plsc.load_gather(
                    cof2_v, [jnp.full((16,), j * CK + r, dtype=i32)])
                for u in range(8):
                    rows_v[b][r, pl.ds(u * 16, 16)] = (
                        rows_v[b][r, pl.ds(u * 16, 16)] * cs)
                return 0

            lax.fori_loop(0, CK, rbody, 0)

        descs_g = [None] * NB
        descs_s = [None] * NB
        for j in range(NSUB):
            b = j % NB
            if descs_s[b] is not None:
                descs_s[b].wait()
            descs_g[b] = pltpu.async_copy(
                esrc.at[src2_v.at[j]], rows_v[b], sems_g[b])
            if j >= 1:
                pb = (j - 1) % NB
                descs_g[pb].wait()
                scale(j - 1)
                descs_s[pb] = pltpu.async_copy(
                    rows_v[pb], acc.at[dst2_v.at[j - 1]], sems_s[pb],
                    add=True)
        lb = (NSUB - 1) % NB
        descs_g[lb].wait()
        scale(NSUB - 1)
        descs_s[lb] = pltpu.async_copy(
            rows_v[lb], acc.at[dst2_v.at[NSUB - 1]], sems_s[lb], add=True)
        for b in range(NB):
            if descs_s[b] is not None:
                descs_s[b].wait()
        return 0

    lax.fori_loop(0, NSUP, sbody, 0)
    plsc.subcore_barrier()
    pltpu.sync_copy(acc.at[pl.ds(sid * ROWS_PT, ROWS_PT)],
                    p_out.at[cid].at[pl.ds(sid * ROWS_PT, ROWS_PT)])


def _combine_body(p, e_out, bufa, bufb, sem):
    cid = lax.axis_index("c")
    sid = lax.axis_index("s")
    wid = sid * NC + cid

    def do(off, n):
        da = pltpu.async_copy(p.at[0].at[pl.ds(off, n)],
                              bufa.at[pl.ds(0, n)], sem)
        db = pltpu.async_copy(p.at[1].at[pl.ds(off, n)],
                              bufb.at[pl.ds(0, n)], sem)
        da.wait(); db.wait()

        def abody(r, _):
            for u in range(8):
                bufa[r, pl.ds(u * 16, 16)] = (
                    bufa[r, pl.ds(u * 16, 16)] + bufb[r, pl.ds(u * 16, 16)])
            return 0

        lax.fori_loop(0, n, abody, 0)
        pltpu.sync_copy(bufa.at[pl.ds(0, n)], e_out.at[pl.ds(off, n)])

    pl.when(wid < 16)(lambda: do(wid * CW_HI, CW_HI))
    pl.when(wid >= 16)(
        lambda: do(16 * CW_HI + (wid - 16) * CW_LO, CW_LO))


def _output_body(oidx_all, emb, e1, e2, p3, out_all,
                 oidx_v, orow_v, oacc_v, sem):
    cid = lax.axis_index("c")
    sid = lax.axis_index("s")
    wid = sid * NC + cid

    def obody(q, _):
        ob = wid * OPT + q * OCK
        pltpu.sync_copy(oidx_all.at[pl.ds(ob, OCK)], oidx_v)
        pltpu.async_copy(emb.at[oidx_v], oacc_v, sem).wait()
        for tbl in (e1, e2, p3.at[0], p3.at[1]):
            pltpu.async_copy(tbl.at[oidx_v], orow_v, sem).wait()

            def abody(j, _):
                for u in range(8):
                    oacc_v[j, pl.ds(u * 16, 16)] = (
                        oacc_v[j, pl.ds(u * 16, 16)]
                        + orow_v[j, pl.ds(u * 16, 16)])
                return 0

            lax.fori_loop(0, OCK, abody, 0)

        def sbody(j, _):
            for u in range(8):
                oacc_v[j, pl.ds(u * 16, 16)] = (
                    oacc_v[j, pl.ds(u * 16, 16)] * 0.25)
            return 0

        lax.fori_loop(0, OCK, sbody, 0)
        pltpu.sync_copy(oacc_v, out_all.at[pl.ds(ob, OCK)])
        return 0

    lax.fori_loop(0, OPT // OCK, obody, 0)


_scatter_call = pl.kernel(
    _scatter_body,
    out_type=jax.ShapeDtypeStruct((NC, N_NODE, D), f32),
    mesh=_mesh,
    scratch_types=(
        pltpu.VMEM_SHARED((N_NODE, D), f32),
        pltpu.VMEM((NSUB, CK), i32),
        pltpu.VMEM((NSUB, CK), i32),
        pltpu.VMEM((SCM,), f32),
        pltpu.VMEM((SCM,), i32),
        pltpu.VMEM((SCM,), f32),
        tuple(pltpu.VMEM((CK, D), f32) for _ in range(NB)),
        pltpu.VMEM((ZR, D), f32),
        pltpu.VMEM((32,), f32),
        pltpu.SemaphoreType.DMA,
        tuple(pltpu.SemaphoreType.DMA for _ in range(NB)),
        tuple(pltpu.SemaphoreType.DMA for _ in range(NB)),
    ),
    compiler_params=_params,
)

_combine_call = pl.kernel(
    _combine_body,
    out_type=jax.ShapeDtypeStruct((N_NODE, D), f32),
    mesh=_mesh,
    scratch_types=(
        pltpu.VMEM((CW_HI, D), f32),
        pltpu.VMEM((CW_HI, D), f32),
        pltpu.SemaphoreType.DMA,
    ),
    compiler_params=_params,
)

_output_call = pl.kernel(
    _output_body,
    out_type=jax.ShapeDtypeStruct((NOUT, D), f32),
    mesh=_mesh,
    scratch_types=(
        pltpu.VMEM((OCK,), i32),
        pltpu.VMEM((OCK, D), f32),
        pltpu.VMEM((OCK, D), f32),
        pltpu.SemaphoreType.DMA,
    ),
    compiler_params=_params,
)


@jax.jit
def kernel(user_idx, item_idx, embedding, alpha_head2tail, alpha_tail2head,
           edge_index, edge_type, edge_norm):
    h_arr = edge_index[0]
    t_arr = edge_index[1]
    src2d = jnp.concatenate([t_arr, h_arr]).reshape(NM // CK, CK)
    dst2d = jnp.concatenate([h_arr, t_arr]).reshape(NM // CK, CK)
    typall = jnp.concatenate([edge_type, edge_type])
    nrmall = jnp.concatenate([edge_norm, edge_norm])
    acat = jnp.concatenate([alpha_tail2head, alpha_head2tail])
    oidx_all = jnp.concatenate([user_idx, item_idx + N_USER])

    p1 = _scatter_call(embedding, src2d, dst2d, typall, nrmall, acat)
    e1 = _combine_call(p1)
    p2 = _scatter_call(e1, src2d, dst2d, typall, nrmall, acat)
    e2 = _combine_call(p2)
    p3 = _scatter_call(e2, src2d, dst2d, typall, nrmall, acat)
    out_all = _output_call(oidx_all, embedding, e1, e2, p3)
    return out_all[:4096], out_all[4096:]


# speedup vs baseline: 8.5811x; 1.1435x over previous
import jax
import jax.numpy as jnp
from jax import lax
from jax.experimental import pallas as pl
from jax.experimental.pallas import tpu as pltpu
from jax.experimental.pallas import tpu_sc as plsc

N_NODE = 10000
N_USER = 6000
D = 128
NE = 320000
NM = 2 * NE
NTILE = 16
NC = 2
MPT = NE // NTILE
CK = 80
NSUB = 25
SCM = CK * NSUB
NSUP = MPT // SCM
NB = 3
RU = 4
ROWS_PT = N_NODE // NTILE
ZR = 25
NOUT = 8192
OPT = NOUT // (NC * NTILE)
OCK = 32
CW_HI = 313
CW_LO = 312

_params = pltpu.CompilerParams(use_tc_tiling_on_sc=False,
                               needs_layout_passes=False)
_mesh = plsc.VectorSubcoreMesh(core_axis_name="c", subcore_axis_name="s",
                               num_cores=NC)
f32 = jnp.float32
i32 = jnp.int32


def _scatter_body(esrc, src2d, dst2d, typall, nrmall, acat, p_out,
                  acc, src2_v, dst2_v, nrm2_v, typ2_v, cof2_v, rows_v,
                  zer_v, acat_v, sem_m, sems_g, sems_s):
    cid = lax.axis_index("c")
    sid = lax.axis_index("s")
    ebase = cid * NE + sid * MPT
    cbase = cid * (NE // CK) + sid * (MPT // CK)

    pltpu.sync_copy(acat, acat_v)
    aoff = jnp.where(cid == 0, 0, 16).astype(i32)

    for r in range(ZR):
        for u in range(8):
            zer_v[r, pl.ds(u * 16, 16)] = jnp.zeros((16,), f32)

    def zbody(z, _):
        pltpu.sync_copy(zer_v, acc.at[pl.ds(sid * ROWS_PT + z * ZR, ZR)])
        return 0

    lax.fori_loop(0, ROWS_PT // ZR, zbody, 0)
    plsc.subcore_barrier()

    def sbody(s, _):
        mb = ebase + s * SCM
        cb = cbase + s * NSUB
        d1 = pltpu.async_copy(src2d.at[pl.ds(cb, NSUB)], src2_v, sem_m)
        d2 = pltpu.async_copy(dst2d.at[pl.ds(cb, NSUB)], dst2_v, sem_m)
        d3 = pltpu.async_copy(nrmall.at[pl.ds(mb, SCM)], nrm2_v, sem_m)
        d4 = pltpu.async_copy(typall.at[pl.ds(mb, SCM)], typ2_v, sem_m)
        d1.wait(); d2.wait(); d3.wait(); d4.wait()

        def kbody(j, _):
            vt = typ2_v[pl.ds(j * 16, 16)]
            vn = nrm2_v[pl.ds(j * 16, 16)]
            va = plsc.load_gather(acat_v, [vt + aoff])
            cof2_v[pl.ds(j * 16, 16)] = vn * va
            return 0

        lax.fori_loop(0, SCM // 16, kbody, 0)

        def scale(j):
            b = j % NB

            def gbody(g, _):
                cvec = cof2_v[pl.ds(j * CK + g * 16, 16)]

                def hbody(hh, _):
                    for k in range(RU):
                        lane = hh * RU + k
                        r = g * 16 + lane
                        cs = jnp.take_along_axis(
                            cvec, jnp.full((16,), lane, dtype=i32), axis=0,
                            mode="promise_in_bounds")
                        for u in range(8):
                            rows_v[b][r, pl.ds(u * 16, 16)] = (
                                rows_v[b][r, pl.ds(u * 16, 16)] * cs)
                    return 0

                lax.fori_loop(0, 16 // RU, hbody, 0)
                return 0

            lax.fori_loop(0, CK // 16, gbody, 0)

        descs_g = [None] * NB
        descs_s = [None] * NB
        for j in range(NSUB):
            b = j % NB
            if descs_s[b] is not None:
                descs_s[b].wait()
            descs_g[b] = pltpu.async_copy(
                esrc.at[src2_v.at[j]], rows_v[b], sems_g[b])
            if j >= 1:
                pb = (j - 1) % NB
                descs_g[pb].wait()
                scale(j - 1)
                descs_s[pb] = pltpu.async_copy(
                    rows_v[pb], acc.at[dst2_v.at[j - 1]], sems_s[pb],
                    add=True)
        lb = (NSUB - 1) % NB
        descs_g[lb].wait()
        scale(NSUB - 1)
        descs_s[lb] = pltpu.async_copy(
            rows_v[lb], acc.at[dst2_v.at[NSUB - 1]], sems_s[lb], add=True)
        for b in range(NB):
            if descs_s[b] is not None:
                descs_s[b].wait()
        return 0

    lax.fori_loop(0, NSUP, sbody, 0)
    plsc.subcore_barrier()
    pltpu.sync_copy(acc.at[pl.ds(sid * ROWS_PT, ROWS_PT)],
                    p_out.at[cid].at[pl.ds(sid * ROWS_PT, ROWS_PT)])


def _combine_body(p, e_out, bufa, bufb, sem):
    cid = lax.axis_index("c")
    sid = lax.axis_index("s")
    wid = sid * NC + cid

    def do(off, n):
        da = pltpu.async_copy(p.at[0].at[pl.ds(off, n)],
                              bufa.at[pl.ds(0, n)], sem)
        db = pltpu.async_copy(p.at[1].at[pl.ds(off, n)],
                              bufb.at[pl.ds(0, n)], sem)
        da.wait(); db.wait()

        def abody(r, _):
            for u in range(8):
                bufa[r, pl.ds(u * 16, 16)] = (
                    bufa[r, pl.ds(u * 16, 16)] + bufb[r, pl.ds(u * 16, 16)])
            return 0

        lax.fori_loop(0, n, abody, 0)
        pltpu.sync_copy(bufa.at[pl.ds(0, n)], e_out.at[pl.ds(off, n)])

    pl.when(wid < 16)(lambda: do(wid * CW_HI, CW_HI))
    pl.when(wid >= 16)(
        lambda: do(16 * CW_HI + (wid - 16) * CW_LO, CW_LO))


def _output_body(oidx_all, emb, e1, e2, p3, out_all,
                 oidx_v, orow_v, oacc_v, sem):
    cid = lax.axis_index("c")
    sid = lax.axis_index("s")
    wid = sid * NC + cid

    def obody(q, _):
        ob = wid * OPT + q * OCK
        pltpu.sync_copy(oidx_all.at[pl.ds(ob, OCK)], oidx_v)
        pltpu.async_copy(emb.at[oidx_v], oacc_v, sem).wait()
        for tbl in (e1, e2, p3.at[0], p3.at[1]):
            pltpu.async_copy(tbl.at[oidx_v], orow_v, sem).wait()

            def abody(j, _):
                for u in range(8):
                    oacc_v[j, pl.ds(u * 16, 16)] = (
                        oacc_v[j, pl.ds(u * 16, 16)]
                        + orow_v[j, pl.ds(u * 16, 16)])
                return 0

            lax.fori_loop(0, OCK, abody, 0)

        def sbody(j, _):
            for u in range(8):
                oacc_v[j, pl.ds(u * 16, 16)] = (
                    oacc_v[j, pl.ds(u * 16, 16)] * 0.25)
            return 0

        lax.fori_loop(0, OCK, sbody, 0)
        pltpu.sync_copy(oacc_v, out_all.at[pl.ds(ob, OCK)])
        return 0

    lax.fori_loop(0, OPT // OCK, obody, 0)


_scatter_call = pl.kernel(
    _scatter_body,
    out_type=jax.ShapeDtypeStruct((NC, N_NODE, D), f32),
    mesh=_mesh,
    scratch_types=(
        pltpu.VMEM_SHARED((N_NODE, D), f32),
        pltpu.VMEM((NSUB, CK), i32),
        pltpu.VMEM((NSUB, CK), i32),
        pltpu.VMEM((SCM,), f32),
        pltpu.VMEM((SCM,), i32),
        pltpu.VMEM((SCM,), f32),
        tuple(pltpu.VMEM((CK, D), f32) for _ in range(NB)),
        pltpu.VMEM((ZR, D), f32),
        pltpu.VMEM((32,), f32),
        pltpu.SemaphoreType.DMA,
        tuple(pltpu.SemaphoreType.DMA for _ in range(NB)),
        tuple(pltpu.SemaphoreType.DMA for _ in range(NB)),
    ),
    compiler_params=_params,
)

_combine_call = pl.kernel(
    _combine_body,
    out_type=jax.ShapeDtypeStruct((N_NODE, D), f32),
    mesh=_mesh,
    scratch_types=(
        pltpu.VMEM((CW_HI, D), f32),
        pltpu.VMEM((CW_HI, D), f32),
        pltpu.SemaphoreType.DMA,
    ),
    compiler_params=_params,
)

_output_call = pl.kernel(
    _output_body,
    out_type=jax.ShapeDtypeStruct((NOUT, D), f32),
    mesh=_mesh,
    scratch_types=(
        pltpu.VMEM((OCK,), i32),
        pltpu.VMEM((OCK, D), f32),
        pltpu.VMEM((OCK, D), f32),
        pltpu.SemaphoreType.DMA,
    ),
    compiler_params=_params,
)


@jax.jit
def kernel(user_idx, item_idx, embedding, alpha_head2tail, alpha_tail2head,
           edge_index, edge_type, edge_norm):
    h_arr = edge_index[0]
    t_arr = edge_index[1]
    src2d = jnp.concatenate([t_arr, h_arr]).reshape(NM // CK, CK)
    dst2d = jnp.concatenate([h_arr, t_arr]).reshape(NM // CK, CK)
    typall = jnp.concatenate([edge_type, edge_type])
    nrmall = jnp.concatenate([edge_norm, edge_norm])
    acat = jnp.concatenate([alpha_tail2head, alpha_head2tail])
    oidx_all = jnp.concatenate([user_idx, item_idx + N_USER])

    p1 = _scatter_call(embedding, src2d, dst2d, typall, nrmall, acat)
    e1 = _combine_call(p1)
    p2 = _scatter_call(e1, src2d, dst2d, typall, nrmall, acat)
    e2 = _combine_call(p2)
    p3 = _scatter_call(e2, src2d, dst2d, typall, nrmall, acat)
    out_all = _output_call(oidx_all, embedding, e1, e2, p3)
    return out_all[:4096], out_all[4096:]


# speedup vs baseline: 8.7823x; 1.0235x over previous
import jax
import jax.numpy as jnp
from jax import lax
from jax.experimental import pallas as pl
from jax.experimental.pallas import tpu as pltpu
from jax.experimental.pallas import tpu_sc as plsc

N_NODE = 10000
N_USER = 6000
D = 128
NE = 320000
NM = 2 * NE
NTILE = 16
NC = 2
MPT = NE // NTILE
CK = 80
NSUB = 25
SCM = CK * NSUB
NSUP = MPT // SCM
SPB = 2
NBLK = NSUP // SPB
NB = 3
RU = 4
ROWS_PT = N_NODE // NTILE
NOUT = 8192
OPT = NOUT // (NC * NTILE)
OCK = 32
CW_HI = 313
CW_LO = 312

_params = pltpu.CompilerParams(use_tc_tiling_on_sc=False,
                               needs_layout_passes=False)
_mesh = plsc.VectorSubcoreMesh(core_axis_name="c", subcore_axis_name="s",
                               num_cores=NC)
f32 = jnp.float32
i32 = jnp.int32


def _scatter_body(esrc, src2d, dst2d, typall, nrmall, acat, zrows, p_out,
                  acc, src2_v, dst2_v, nrm2_v, typ2_v, cof2_v, rows_v,
                  acat_v, sems_m, sems_g, sems_s):
    cid = lax.axis_index("c")
    sid = lax.axis_index("s")
    ebase = cid * NE + sid * MPT
    cbase = cid * (NE // CK) + sid * (MPT // CK)

    pltpu.sync_copy(acat, acat_v)
    aoff = jnp.where(cid == 0, 0, 16).astype(i32)

    pltpu.sync_copy(zrows, acc.at[pl.ds(sid * ROWS_PT, ROWS_PT)])
    plsc.subcore_barrier()

    def issue_meta(sn, st):
        mb = ebase + sn * SCM
        cb = cbase + sn * NSUB
        pltpu.async_copy(src2d.at[pl.ds(cb, NSUB)], src2_v[st], sems_m[st])
        pltpu.async_copy(dst2d.at[pl.ds(cb, NSUB)], dst2_v[st], sems_m[st])
        pltpu.async_copy(nrmall.at[pl.ds(mb, SCM)], nrm2_v[st], sems_m[st])
        pltpu.async_copy(typall.at[pl.ds(mb, SCM)], typ2_v[st], sems_m[st])

    def wait_meta(st):
        for _ in range(4):
            pltpu.make_async_copy(nrmall.at[pl.ds(0, SCM)], nrm2_v[st],
                                  sems_m[st]).wait()

    issue_meta(0, 0)
    issue_meta(1, 1)

    def bbody(bb, _):
        descs_g = [None] * NB
        descs_s = [None] * NB
        for s2 in range(SPB):
            s = bb * SPB + s2
            wait_meta(s2)
            issue_meta(lax.rem(s + SPB, NSUP), s2)

            def kbody(j, _, s2=s2):
                vt = typ2_v[s2][pl.ds(j * 16, 16)]
                vn = nrm2_v[s2][pl.ds(j * 16, 16)]
                va = plsc.load_gather(acat_v, [vt + aoff])
                cof2_v[pl.ds(j * 16, 16)] = vn * va
                return 0

            lax.fori_loop(0, SCM // 16, kbody, 0)

            def scale(j, b, s2=s2):
                def gbody(g, _):
                    cvec = cof2_v[pl.ds(j * CK + g * 16, 16)]

                    def hbody(hh, _):
                        for k in range(RU):
                            lane = hh * RU + k
                            r = g * 16 + lane
                            cs = jnp.take_along_axis(
                                cvec, jnp.full((16,), lane, dtype=i32),
                                axis=0, mode="promise_in_bounds")
                            for u in range(8):
                                rows_v[b][r, pl.ds(u * 16, 16)] = (
                                    rows_v[b][r, pl.ds(u * 16, 16)] * cs)
                        return 0

                    lax.fori_loop(0, 16 // RU, hbody, 0)
                    return 0

                lax.fori_loop(0, CK // 16, gbody, 0)

            for j in range(NSUB):
                jj = s2 * NSUB + j
                b = jj % NB
                if descs_s[b] is not None:
                    descs_s[b].wait()
                descs_g[b] = pltpu.async_copy(
                    esrc.at[src2_v[s2].at[j]], rows_v[b], sems_g[b])
                if jj >= 1:
                    pj = jj - 1
                    pb = pj % NB
                    ps = pj // NSUB
                    descs_g[pb].wait()
                    scale(pj - ps * NSUB, pb, s2=ps)
                    descs_s[pb] = pltpu.async_copy(
                        rows_v[pb],
                        acc.at[dst2_v[ps].at[pj - ps * NSUB]],
                        sems_s[pb], add=True)
        lj = SPB * NSUB - 1
        lb = lj % NB
        descs_g[lb].wait()
        scale(lj - (SPB - 1) * NSUB, lb, s2=SPB - 1)
        descs_s[lb] = pltpu.async_copy(
            rows_v[lb], acc.at[dst2_v[SPB - 1].at[NSUB - 1]], sems_s[lb],
            add=True)
        for b in range(NB):
            if descs_s[b] is not None:
                descs_s[b].wait()
        return 0

    lax.fori_loop(0, NBLK, bbody, 0)
    wait_meta(0)
    wait_meta(1)
    plsc.subcore_barrier()
    pltpu.sync_copy(acc.at[pl.ds(sid * ROWS_PT, ROWS_PT)],
                    p_out.at[cid].at[pl.ds(sid * ROWS_PT, ROWS_PT)])


def _combine_body(p, e_out, bufa, bufb, sem):
    cid = lax.axis_index("c")
    sid = lax.axis_index("s")
    wid = sid * NC + cid

    def do(off, n):
        da = pltpu.async_copy(p.at[0].at[pl.ds(off, n)],
                              bufa.at[pl.ds(0, n)], sem)
        db = pltpu.async_copy(p.at[1].at[pl.ds(off, n)],
                              bufb.at[pl.ds(0, n)], sem)
        da.wait(); db.wait()

        def abody(r, _):
            for u in range(8):
                bufa[r, pl.ds(u * 16, 16)] = (
                    bufa[r, pl.ds(u * 16, 16)] + bufb[r, pl.ds(u * 16, 16)])
            return 0

        lax.fori_loop(0, n, abody, 0)
        pltpu.sync_copy(bufa.at[pl.ds(0, n)], e_out.at[pl.ds(off, n)])

    pl.when(wid < 16)(lambda: do(wid * CW_HI, CW_HI))
    pl.when(wid >= 16)(
        lambda: do(16 * CW_HI + (wid - 16) * CW_LO, CW_LO))


def _output_body(oidx_all, emb, e1, e2, p3, out_all,
                 oidx_v, orow_v, oacc_v, sem):
    cid = lax.axis_index("c")
    sid = lax.axis_index("s")
    wid = sid * NC + cid

    def obody(q, _):
        ob = wid * OPT + q * OCK
        pltpu.sync_copy(oidx_all.at[pl.ds(ob, OCK)], oidx_v)
        pltpu.async_copy(emb.at[oidx_v], oacc_v, sem).wait()
        for tbl in (e1, e2, p3.at[0], p3.at[1]):
            pltpu.async_copy(tbl.at[oidx_v], orow_v, sem).wait()

            def abody(j, _):
                for u in range(8):
                    oacc_v[j, pl.ds(u * 16, 16)] = (
                        oacc_v[j, pl.ds(u * 16, 16)]
                        + orow_v[j, pl.ds(u * 16, 16)])
                return 0

            lax.fori_loop(0, OCK, abody, 0)

        def sbody(j, _):
            for u in range(8):
                oacc_v[j, pl.ds(u * 16, 16)] = (
                    oacc_v[j, pl.ds(u * 16, 16)] * 0.25)
            return 0

        lax.fori_loop(0, OCK, sbody, 0)
        pltpu.sync_copy(oacc_v, out_all.at[pl.ds(ob, OCK)])
        return 0

    lax.fori_loop(0, OPT // OCK, obody, 0)


_scatter_call = pl.kernel(
    _scatter_body,
    out_type=jax.ShapeDtypeStruct((NC, N_NODE, D), f32),
    mesh=_mesh,
    scratch_types=(
        pltpu.VMEM_SHARED((N_NODE, D), f32),
        tuple(pltpu.VMEM((NSUB, CK), i32) for _ in range(SPB)),
        tuple(pltpu.VMEM((NSUB, CK), i32) for _ in range(SPB)),
        tuple(pltpu.VMEM((SCM,), f32) for _ in range(SPB)),
        tuple(pltpu.VMEM((SCM,), i32) for _ in range(SPB)),
        pltpu.VMEM((SCM,), f32),
        tuple(pltpu.VMEM((CK, D), f32) for _ in range(NB)),
        pltpu.VMEM((32,), f32),
        tuple(pltpu.SemaphoreType.DMA for _ in range(SPB)),
        tuple(pltpu.SemaphoreType.DMA for _ in range(NB)),
        tuple(pltpu.SemaphoreType.DMA for _ in range(NB)),
    ),
    compiler_params=_params,
)

_combine_call = pl.kernel(
    _combine_body,
    out_type=jax.ShapeDtypeStruct((N_NODE, D), f32),
    mesh=_mesh,
    scratch_types=(
        pltpu.VMEM((CW_HI, D), f32),
        pltpu.VMEM((CW_HI, D), f32),
        pltpu.SemaphoreType.DMA,
    ),
    compiler_params=_params,
)

_output_call = pl.kernel(
    _output_body,
    out_type=jax.ShapeDtypeStruct((NOUT, D), f32),
    mesh=_mesh,
    scratch_types=(
        pltpu.VMEM((OCK,), i32),
        pltpu.VMEM((OCK, D), f32),
        pltpu.VMEM((OCK, D), f32),
        pltpu.SemaphoreType.DMA,
    ),
    compiler_params=_params,
)


@jax.jit
def kernel(user_idx, item_idx, embedding, alpha_head2tail, alpha_tail2head,
           edge_index, edge_type, edge_norm):
    h_arr = edge_index[0]
    t_arr = edge_index[1]
    src2d = jnp.concatenate([t_arr, h_arr]).reshape(NM // CK, CK)
    dst2d = jnp.concatenate([h_arr, t_arr]).reshape(NM // CK, CK)
    typall = jnp.concatenate([edge_type, edge_type])
    nrmall = jnp.concatenate([edge_norm, edge_norm])
    acat = jnp.concatenate([alpha_tail2head, alpha_head2tail])
    oidx_all = jnp.concatenate([user_idx, item_idx + N_USER])
    zrows = jnp.zeros((ROWS_PT, D), f32)

    p1 = _scatter_call(embedding, src2d, dst2d, typall, nrmall, acat, zrows)
    e1 = _combine_call(p1)
    p2 = _scatter_call(e1, src2d, dst2d, typall, nrmall, acat, zrows)
    e2 = _combine_call(p2)
    p3 = _scatter_call(e2, src2d, dst2d, typall, nrmall, acat, zrows)
    out_all = _output_call(oidx_all, embedding, e1, e2, p3)
    return out_all[:4096], out_all[4096:]
